# speedup vs baseline: 3.0246x; 3.0246x over previous
import functools

import jax
import jax.numpy as jnp
from jax import lax
from jax.experimental import pallas as pl
from jax.experimental.pallas import tpu as pltpu
from jax.experimental.pallas import tpu_sc as plsc

NUM_WORKERS = 32
IDX_CHUNK = 128


def _gather_pair(uid2, iid2, user_table, item_table):
    n_rows_total, chunk = uid2.shape
    batch = n_rows_total * chunk
    embed = user_table.shape[1]
    rows_per_w = batch // NUM_WORKERS
    nch = rows_per_w // chunk

    mesh = plsc.VectorSubcoreMesh(core_axis_name="c", subcore_axis_name="s")

    @functools.partial(
        pl.kernel,
        mesh=mesh,
        out_type=(
            jax.ShapeDtypeStruct((batch, embed), jnp.float32),
            jax.ShapeDtypeStruct((batch, embed), jnp.float32),
        ),
        scratch_types=[
            pltpu.VMEM((nch, chunk), jnp.int32),
            pltpu.VMEM((nch, chunk), jnp.int32),
            pltpu.VMEM((rows_per_w, embed), jnp.float32),
            pltpu.SemaphoreType.DMA,
        ],
    )
    def gather_kernel(uid_hbm, iid_hbm, ut_hbm, it_hbm, out_u, out_i,
                      uidx_v, iidx_v, rows_v, sem):
        wid = lax.axis_index("s") * 2 + lax.axis_index("c")
        base = wid * rows_per_w
        idx_row = wid * nch
        pltpu.sync_copy(uid_hbm.at[pl.ds(idx_row, nch)], uidx_v)
        pltpu.sync_copy(iid_hbm.at[pl.ds(idx_row, nch)], iidx_v)
        cps = [
            pltpu.async_copy(ut_hbm.at[uidx_v.at[j]],
                             rows_v.at[pl.ds(j * chunk, chunk)], sem)
            for j in range(nch)
        ]
        for c in cps:
            c.wait()
        pltpu.sync_copy(rows_v, out_u.at[pl.ds(base, rows_per_w)])
        cps = [
            pltpu.async_copy(it_hbm.at[iidx_v.at[j]],
                             rows_v.at[pl.ds(j * chunk, chunk)], sem)
            for j in range(nch)
        ]
        for c in cps:
            c.wait()
        pltpu.sync_copy(rows_v, out_i.at[pl.ds(base, rows_per_w)])

    return gather_kernel(uid2, iid2, user_table, item_table)


def _mlp_body(ue_ref, ie_ref, w1u_ref, w1i_ref, b1_ref, w2_ref, b2_ref,
              w3_ref, b3_ref, o_ref):
    h = jnp.dot(ue_ref[...], w1u_ref[...], preferred_element_type=jnp.float32)
    h = h + jnp.dot(ie_ref[...], w1i_ref[...],
                    preferred_element_type=jnp.float32)
    h1 = jnp.maximum(h + b1_ref[...], 0.0)
    h2 = jnp.maximum(
        jnp.dot(h1, w2_ref[...], preferred_element_type=jnp.float32)
        + b2_ref[...], 0.0)
    o = jnp.maximum(
        jnp.dot(h2, w3_ref[...], preferred_element_type=jnp.float32)
        + b3_ref[...], 0.0)
    o_ref[...] = o[:, 0]


def _mlp(ue, ie, w1u, w1i, b1, w2, b2, w3, b3, block=2048):
    batch, embed = ue.shape
    grid = batch // block
    full = lambda shape: pl.BlockSpec(shape, lambda i: (0, 0))
    return pl.pallas_call(
        _mlp_body,
        grid=(grid,),
        in_specs=[
            pl.BlockSpec((block, embed), lambda i: (i, 0)),
            pl.BlockSpec((block, embed), lambda i: (i, 0)),
            full(w1u.shape),
            full(w1i.shape),
            full(b1.shape),
            full(w2.shape),
            full(b2.shape),
            full(w3.shape),
            full(b3.shape),
        ],
        out_specs=pl.BlockSpec((block,), lambda i: (i,)),
        out_shape=jax.ShapeDtypeStruct((batch,), jnp.float32),
    )(ue, ie, w1u, w1i, b1, w2, b2, w3, b3)


def kernel(user_id, item_id, user_table, item_table, W1, b1, W2, b2, W3, b3):
    batch = user_id.shape[0]
    embed = user_table.shape[1]
    uid2 = user_id.astype(jnp.int32).reshape(batch // IDX_CHUNK, IDX_CHUNK)
    iid2 = item_id.astype(jnp.int32).reshape(batch // IDX_CHUNK, IDX_CHUNK)
    ue, ie = _gather_pair(uid2, iid2, user_table, item_table)
    w1u = W1[:embed]
    w1i = W1[embed:]
    return _mlp(ue, ie, w1u, w1i, b1.reshape(1, -1), W2, b2.reshape(1, -1),
                W3, b3.reshape(1, 1))
